# Initial kernel scaffold; baseline (speedup 1.0000x reference)
import jax
import jax.numpy as jnp
from jax.experimental import pallas as pl


def kernel(objectness, box_regression, anchors_rrects):
    raise NotImplementedError("write your pallas kernel here")


# speedup vs baseline: 3.3539x; 3.3539x over previous
import numpy as np
import jax
import jax.numpy as jnp
from jax.experimental import pallas as pl
from jax.experimental.pallas import tpu as pltpu

_PRE_N = 2000
_POST_N = 1000
_THRESH = 0.7
_C = 5
_LANES = 128
_CAND_ROWS = 16


def _make_body(n_valid, rows):
    flat_big = np.int32(1 << 30)
    clipv = np.float32(np.log(1000.0 / 16.0))

    def body(obj_ref, breg_ref, anch_ref, out_ref,
             score_ref, src_ref, cand_ref, sup_ref, count_ref, kidx_ref):
        ri = jax.lax.broadcasted_iota(jnp.int32, (rows, _LANES), 0)
        ci = jax.lax.broadcasted_iota(jnp.int32, (rows, _LANES), 1)
        flat = ri * _LANES + ci
        sig = jax.nn.sigmoid(obj_ref[0])
        score_ref[...] = jnp.where(flat < n_valid, sig, -1.0)

        dx = breg_ref[0, 0]
        dy = breg_ref[0, 1]
        dw = jnp.clip(breg_ref[0, 2], -clipv, clipv)
        dh = jnp.clip(breg_ref[0, 3], -clipv, clipv)
        dt = breg_ref[0, 4]
        xa = anch_ref[0, 0]
        ya = anch_ref[0, 1]
        wa = anch_ref[0, 2]
        ha = anch_ref[0, 3]
        ta = anch_ref[0, 4]
        xc = dx * wa + xa
        yc = dy * ha + ya
        w = wa * jnp.exp(dw)
        h = ha * jnp.exp(dh)
        th = dt * np.float32(180.0 / np.pi) + ta
        rad = th * np.float32(np.pi / 180.0)
        cs = jnp.abs(jnp.cos(rad))
        sn = jnp.abs(jnp.sin(rad))
        bw = w * cs + h * sn
        bh = w * sn + h * cs
        x1 = xc - bw / 2
        y1 = yc - bh / 2
        x2 = xc + bw / 2
        y2 = yc + bh / 2
        area = (x2 - x1) * (y2 - y1)
        src_ref[0] = xc
        src_ref[1] = yc
        src_ref[2] = w
        src_ref[3] = h
        src_ref[4] = th
        src_ref[5] = x1
        src_ref[6] = y1
        src_ref[7] = x2
        src_ref[8] = y2
        src_ref[9] = area

        cflat = (jax.lax.broadcasted_iota(jnp.int32, (_CAND_ROWS, _LANES), 0)
                 * _LANES
                 + jax.lax.broadcasted_iota(jnp.int32, (_CAND_ROWS, _LANES), 1))
        sup_ref[...] = jnp.where(cflat < _PRE_N, 0.0, 1.0)
        cand_ref[...] = jnp.zeros((11, _CAND_ROWS, _LANES), jnp.float32)
        count_ref[0] = 0

        liota = jax.lax.broadcasted_iota(jnp.int32, (1, _LANES), 1)

        def lane_get(row, c):
            return jnp.sum(jnp.where(liota == c, row, 0.0))

        def sel_body(i, carry):
            s = score_ref[...]
            m = jnp.max(s)
            idx = jnp.min(jnp.where(s == m, flat, flat_big))
            r = idx // _LANES
            c = idx - r * _LANES
            srow = score_ref[pl.ds(r, 1), :]
            score_ref[pl.ds(r, 1), :] = jnp.where(liota == c, -1.0, srow)
            cr = i // _LANES
            cc = i - cr * _LANES
            for k in range(10):
                dst = k if k < 5 else k + 1
                v = lane_get(src_ref[k, pl.ds(r, 1), :], c)
                crow = cand_ref[dst, pl.ds(cr, 1), :]
                cand_ref[dst, pl.ds(cr, 1), :] = jnp.where(
                    liota == cc, v, crow)
            crow = cand_ref[5, pl.ds(cr, 1), :]
            cand_ref[5, pl.ds(cr, 1), :] = jnp.where(liota == cc, m, crow)
            return carry

        jax.lax.fori_loop(0, _PRE_N, sel_body, 0)

        def nms_body(i, carry):
            r = i // _LANES
            c = i - r * _LANES
            supv = lane_get(sup_ref[pl.ds(r, 1), :], c)
            cnt = count_ref[0]
            keep = jnp.logical_and(supv == 0.0, cnt < _POST_N)

            @pl.when(keep)
            def _():
                kidx_ref[cnt] = i
                count_ref[0] = cnt + 1
                x1i = lane_get(cand_ref[6, pl.ds(r, 1), :], c)
                y1i = lane_get(cand_ref[7, pl.ds(r, 1), :], c)
                x2i = lane_get(cand_ref[8, pl.ds(r, 1), :], c)
                y2i = lane_get(cand_ref[9, pl.ds(r, 1), :], c)
                ai = lane_get(cand_ref[10, pl.ds(r, 1), :], c)
                xx1 = jnp.maximum(cand_ref[6], x1i)
                yy1 = jnp.maximum(cand_ref[7], y1i)
                xx2 = jnp.minimum(cand_ref[8], x2i)
                yy2 = jnp.minimum(cand_ref[9], y2i)
                iw = jnp.maximum(xx2 - xx1, 0.0)
                ih = jnp.maximum(yy2 - yy1, 0.0)
                inter = iw * ih
                iou = inter / (ai + cand_ref[10] - inter + 1e-9)
                sup_ref[...] = jnp.maximum(
                    sup_ref[...], jnp.where(iou > _THRESH, 1.0, 0.0))

            return carry

        jax.lax.fori_loop(0, _PRE_N, nms_body, 0)

        cnt = count_ref[0]
        lastslot = kidx_ref[cnt - 1]

        def pad_body(j, carry):
            @pl.when(j >= cnt)
            def _():
                kidx_ref[j] = lastslot
            return carry

        jax.lax.fori_loop(0, _POST_N, pad_body, 0)

        oiota = jax.lax.broadcasted_iota(jnp.int32, (1, 6), 1)

        def out_body(j, carry):
            slot = kidx_ref[j]
            r = slot // _LANES
            c = slot - r * _LANES
            row = jnp.zeros((1, 6), jnp.float32)
            for k in range(6):
                v = lane_get(cand_ref[k, pl.ds(r, 1), :], c)
                row = jnp.where(oiota == k, v, row)
            out_ref[0, pl.ds(j, 1), :] = row
            return carry

        jax.lax.fori_loop(0, _POST_N, out_body, 0)

    return body


def kernel(objectness, box_regression, anchors_rrects):
    N, A, H, W = objectness.shape
    nA = A * H * W
    rows = ((nA + _LANES - 1) // _LANES + 7) // 8 * 8
    padded = rows * _LANES
    pad = padded - nA

    obj = objectness.reshape(N, A, 1, H, W).transpose(0, 3, 4, 1, 2)
    obj = obj.reshape(N, nA)
    breg = box_regression.reshape(N, A, _C, H, W).transpose(0, 3, 4, 1, 2)
    breg = breg.reshape(N, nA, _C)

    obj_p = jnp.pad(obj, ((0, 0), (0, pad))).reshape(N, rows, _LANES)
    breg_t = jnp.pad(breg.transpose(0, 2, 1),
                     ((0, 0), (0, 0), (0, pad))).reshape(N, _C, rows, _LANES)
    anch_t = jnp.pad(anchors_rrects.transpose(0, 2, 1),
                     ((0, 0), (0, 0), (0, pad))).reshape(N, _C, rows, _LANES)

    out = pl.pallas_call(
        _make_body(nA, rows),
        grid=(N,),
        in_specs=[
            pl.BlockSpec((1, rows, _LANES), lambda n: (n, 0, 0)),
            pl.BlockSpec((1, _C, rows, _LANES), lambda n: (n, 0, 0, 0)),
            pl.BlockSpec((1, _C, rows, _LANES), lambda n: (n, 0, 0, 0)),
        ],
        out_specs=pl.BlockSpec((1, _POST_N, 6), lambda n: (n, 0, 0)),
        out_shape=jax.ShapeDtypeStruct((N, _POST_N, 6), jnp.float32),
        scratch_shapes=[
            pltpu.VMEM((rows, _LANES), jnp.float32),
            pltpu.VMEM((10, rows, _LANES), jnp.float32),
            pltpu.VMEM((11, _CAND_ROWS, _LANES), jnp.float32),
            pltpu.VMEM((_CAND_ROWS, _LANES), jnp.float32),
            pltpu.SMEM((1,), jnp.int32),
            pltpu.SMEM((_POST_N,), jnp.int32),
        ],
    )(obj_p, breg_t, anch_t)
    return out
